# baseline (speedup 1.0000x reference)
import jax
import jax.numpy as jnp
from jax.experimental import pallas as pl
from jax.experimental.pallas import tpu as pltpu

_N = 16384
_D = 128


def _partner(x, j, low):
    if j < _D:
        lo = pltpu.roll(x, _D - j, axis=1)
        hi = pltpu.roll(x, j, axis=1)
    else:
        d = j // _D
        lo = pltpu.roll(x, _D - d, axis=0)
        hi = pltpu.roll(x, d, axis=0)
    return jnp.where(low, lo, hi)


def _listmle_body(t_ref, p_ref, upper_ref, strict_ref, out_ref):
    r_iota = jax.lax.broadcasted_iota(jnp.int32, (_D, _D), 0)
    c_iota = jax.lax.broadcasted_iota(jnp.int32, (_D, _D), 1)

    low_masks = {}
    perms = {}
    j = 1
    while j < _N:
        if j < _D:
            low_masks[j] = (c_iota & j) == 0
            perms[j] = jnp.where((r_iota ^ c_iota) == j, 1.0, 0.0)
        else:
            d = j // _D
            low_masks[j] = (r_iota & d) == 0
            perms[j] = jnp.where((r_iota ^ c_iota) == d, 1.0, 0.0)
        j *= 2

    bits = jax.lax.bitcast_convert_type(t_ref[...], jnp.int32)
    key = jnp.where(bits < 0, bits ^ jnp.int32(0x7FFFFFFF), bits)
    val = p_ref[...]

    flip_prev = jnp.zeros((_D, _D), jnp.int32)
    m = 2
    while m <= _N:
        if m < _N:
            if m < _D:
                bit = (c_iota & m) >> m.bit_length() - 1
            else:
                d = m // _D
                bit = (r_iota & d) >> d.bit_length() - 1
            flip = -bit
        else:
            flip = jnp.zeros((_D, _D), jnp.int32)
        key = key ^ (flip ^ flip_prev)
        flip_prev = flip
        j = m // 2
        while j >= 1:
            low = low_masks[j]
            pk = _partner(key, j, low)
            if j < _D:
                pv = jax.lax.dot_general(
                    val, perms[j], (((1,), (0,)), ((), ())),
                    preferred_element_type=jnp.float32,
                    precision=jax.lax.Precision.HIGHEST)
            else:
                pv = jax.lax.dot_general(
                    perms[j], val, (((1,), (0,)), ((), ())),
                    preferred_element_type=jnp.float32,
                    precision=jax.lax.Precision.HIGHEST)
            new_key = jnp.where(low, jnp.minimum(key, pk),
                                jnp.maximum(key, pk))
            kept = new_key == key
            key = new_key
            val = jnp.where(kept, val, pv)
            j //= 2
        m *= 2
    key = key ^ flip_prev

    mx = jnp.max(val)
    shifted_sum = jnp.sum(val) - _N * mx
    e = jnp.exp(val - mx)

    s_intra = jax.lax.dot_general(
        e, upper_ref[...], (((1,), (0,)), ((), ())),
        preferred_element_type=jnp.float32)
    row_tot = jnp.sum(e, axis=1, keepdims=True)
    carry = jax.lax.dot_general(
        strict_ref[...], row_tot, (((1,), (0,)), ((), ())),
        preferred_element_type=jnp.float32)
    s = s_intra + carry
    total = jnp.sum(jnp.log(s), axis=(0, 1), keepdims=True) - shifted_sum
    out_ref[...] = total.reshape(1, 1) / _N


def kernel(pred, target):
    t2 = target.reshape(_D, _D)
    p2 = pred.reshape(_D, _D)
    a = jnp.arange(_D)
    upper = (a[:, None] >= a[None, :]).astype(jnp.float32)
    strict = (a[None, :] > a[:, None]).astype(jnp.float32)
    out = pl.pallas_call(
        _listmle_body,
        out_shape=jax.ShapeDtypeStruct((1, 1), jnp.float32),
    )(t2, p2, upper, strict)
    return out[0, 0]


# speedup vs baseline: 2.5421x; 2.5421x over previous
import jax
import jax.numpy as jnp
from jax.experimental import pallas as pl
from jax.experimental.pallas import tpu as pltpu

_N = 16384
_D = 128


def _partner(x, j, low):
    if j < _D:
        lo = pltpu.roll(x, _D - j, axis=1)
        hi = pltpu.roll(x, j, axis=1)
    else:
        d = j // _D
        lo = pltpu.roll(x, _D - d, axis=0)
        hi = pltpu.roll(x, d, axis=0)
    return jnp.where(low, lo, hi)


def _listmle_body(t_ref, p_ref, upper_ref, strict_ref, out_ref):
    r_iota = jax.lax.broadcasted_iota(jnp.int32, (_D, _D), 0)
    c_iota = jax.lax.broadcasted_iota(jnp.int32, (_D, _D), 1)

    low_masks = {}
    j = 1
    while j < _N:
        low_masks[j] = ((c_iota & j) == 0) if j < _D else \
            ((r_iota & (j // _D)) == 0)
        j *= 2

    bits = jax.lax.bitcast_convert_type(t_ref[...], jnp.int32)
    key = jnp.where(bits < 0, bits ^ jnp.int32(0x7FFFFFFF), bits)
    val = p_ref[...]

    flip_prev = jnp.zeros((_D, _D), jnp.int32)
    m = 2
    while m <= _N:
        if m < _N:
            if m < _D:
                bit = (c_iota & m) >> m.bit_length() - 1
            else:
                d = m // _D
                bit = (r_iota & d) >> d.bit_length() - 1
            flip = -bit
        else:
            flip = jnp.zeros((_D, _D), jnp.int32)
        key = key ^ (flip ^ flip_prev)
        flip_prev = flip
        j = m // 2
        while j >= 1:
            low = low_masks[j]
            pk = _partner(key, j, low)
            pv = _partner(val, j, low)
            new_key = jnp.where(low, jnp.minimum(key, pk),
                                jnp.maximum(key, pk))
            kept = new_key == key
            key = new_key
            val = jnp.where(kept, val, pv)
            j //= 2
        m *= 2
    key = key ^ flip_prev

    mx = jnp.max(val)
    shifted_sum = jnp.sum(val) - _N * mx
    e = jnp.exp(val - mx)

    s_intra = jax.lax.dot_general(
        e, upper_ref[...], (((1,), (0,)), ((), ())),
        preferred_element_type=jnp.float32)
    row_tot = jnp.sum(e, axis=1, keepdims=True)
    carry = jax.lax.dot_general(
        strict_ref[...], row_tot, (((1,), (0,)), ((), ())),
        preferred_element_type=jnp.float32)
    s = s_intra + carry
    total = jnp.sum(jnp.log(s), axis=(0, 1), keepdims=True) - shifted_sum
    out_ref[...] = total.reshape(1, 1) / _N


def kernel(pred, target):
    t2 = target.reshape(_D, _D)
    p2 = pred.reshape(_D, _D)
    a = jnp.arange(_D)
    upper = (a[:, None] >= a[None, :]).astype(jnp.float32)
    strict = (a[None, :] > a[:, None]).astype(jnp.float32)
    out = pl.pallas_call(
        _listmle_body,
        out_shape=jax.ShapeDtypeStruct((1, 1), jnp.float32),
    )(t2, p2, upper, strict)
    return out[0, 0]


# speedup vs baseline: 3.5822x; 1.4092x over previous
import jax
import jax.numpy as jnp
from jax.experimental import pallas as pl
from jax.experimental.pallas import tpu as pltpu

_N = 16384
_D = 128


def _partner(x, j, low):
    if j < _D:
        lo = pltpu.roll(x, _D - j, axis=0)
        hi = pltpu.roll(x, j, axis=0)
    else:
        d = j // _D
        lo = pltpu.roll(x, _D - d, axis=1)
        hi = pltpu.roll(x, d, axis=1)
    return jnp.where(low, lo, hi)


def _listmle_body(t_ref, p_ref, upper_ref, strict_ref, out_ref):
    r_iota = jax.lax.broadcasted_iota(jnp.int32, (_D, _D), 0)
    c_iota = jax.lax.broadcasted_iota(jnp.int32, (_D, _D), 1)

    low_masks = {}
    j = 1
    while j < _N:
        low_masks[j] = ((r_iota & j) == 0) if j < _D else \
            ((c_iota & (j // _D)) == 0)
        j *= 2

    bits = jax.lax.bitcast_convert_type(t_ref[...], jnp.int32)
    key = jnp.where(bits < 0, bits ^ jnp.int32(0x7FFFFFFF), bits)
    val = p_ref[...]

    flip_prev = jnp.zeros((_D, _D), jnp.int32)
    m = 2
    while m <= _N:
        if m < _N:
            if m < _D:
                bit = (r_iota & m) >> m.bit_length() - 1
            else:
                d = m // _D
                bit = (c_iota & d) >> d.bit_length() - 1
            flip = -bit
        else:
            flip = jnp.zeros((_D, _D), jnp.int32)
        key = key ^ (flip ^ flip_prev)
        flip_prev = flip
        j = m // 2
        while j >= 1:
            low = low_masks[j]
            pk = _partner(key, j, low)
            pv = _partner(val, j, low)
            new_key = jnp.where(low, jnp.minimum(key, pk),
                                jnp.maximum(key, pk))
            kept = new_key == key
            key = new_key
            val = jnp.where(kept, val, pv)
            j //= 2
        m *= 2
    key = key ^ flip_prev

    mx = jnp.max(val)
    shifted_sum = jnp.sum(val) - _N * mx
    e = jnp.exp(val - mx)

    s_intra = jax.lax.dot_general(
        upper_ref[...], e, (((1,), (0,)), ((), ())),
        preferred_element_type=jnp.float32)
    col_tot = jnp.sum(e, axis=0, keepdims=True)
    carry = jax.lax.dot_general(
        col_tot, strict_ref[...], (((1,), (0,)), ((), ())),
        preferred_element_type=jnp.float32)
    s = s_intra + carry
    total = jnp.sum(jnp.log(s), axis=(0, 1), keepdims=True) - shifted_sum
    out_ref[...] = total.reshape(1, 1) / _N


def kernel(pred, target):
    t2 = target.reshape(_D, _D)
    p2 = pred.reshape(_D, _D)
    a = jnp.arange(_D)
    upper = (a[None, :] >= a[:, None]).astype(jnp.float32)
    strict = (a[:, None] > a[None, :]).astype(jnp.float32)
    out = pl.pallas_call(
        _listmle_body,
        out_shape=jax.ShapeDtypeStruct((1, 1), jnp.float32),
    )(t2, p2, upper, strict)
    return out[0, 0]


# speedup vs baseline: 3.6036x; 1.0060x over previous
import jax
import jax.numpy as jnp
from jax.experimental import pallas as pl
from jax.experimental.pallas import tpu as pltpu

_N = 16384
_D = 128


def _partner(x, j, low):
    if j < _D:
        lo = pltpu.roll(x, _D - j, axis=0)
        hi = pltpu.roll(x, j, axis=0)
    else:
        d = j // _D
        lo = pltpu.roll(x, _D - d, axis=1)
        hi = pltpu.roll(x, d, axis=1)
    return jnp.where(low, lo, hi)


def _listmle_body(t_ref, p_ref, upper_ref, strict_ref, out_ref):
    r_iota = jax.lax.broadcasted_iota(jnp.int32, (_D, _D), 0)
    c_iota = jax.lax.broadcasted_iota(jnp.int32, (_D, _D), 1)

    low_masks = {}
    j = 1
    while j < _N:
        low_masks[j] = ((r_iota & j) == 0) if j < _D else \
            ((c_iota & (j // _D)) == 0)
        j *= 2

    bits = jax.lax.bitcast_convert_type(t_ref[...], jnp.int32)
    key = jnp.where(bits < 0, bits ^ jnp.int32(0x7FFFFFFF), bits)
    val = p_ref[...]

    flip_prev = jnp.zeros((_D, _D), jnp.int32)
    m = 2
    while m <= _N:
        if m < _N:
            if m < _D:
                bit = (r_iota & m) >> m.bit_length() - 1
            else:
                d = m // _D
                bit = (c_iota & d) >> d.bit_length() - 1
            flip = -bit
        else:
            flip = jnp.zeros((_D, _D), jnp.int32)
        key = key ^ (flip ^ flip_prev)
        flip_prev = flip
        j = m // 2
        while j >= 1:
            low = low_masks[j]
            pk = _partner(key, j, low)
            pv = _partner(val, j, low)
            keep = jnp.logical_xor(low, key >= pk)
            key = jnp.where(keep, key, pk)
            val = jnp.where(keep, val, pv)
            j //= 2
        m *= 2
    key = key ^ flip_prev

    mx = jnp.max(val)
    shifted_sum = jnp.sum(val) - _N * mx
    e = jnp.exp(val - mx)

    s_intra = jax.lax.dot_general(
        upper_ref[...], e, (((1,), (0,)), ((), ())),
        preferred_element_type=jnp.float32)
    col_tot = jnp.sum(e, axis=0, keepdims=True)
    carry = jax.lax.dot_general(
        col_tot, strict_ref[...], (((1,), (0,)), ((), ())),
        preferred_element_type=jnp.float32)
    s = s_intra + carry
    total = jnp.sum(jnp.log(s), axis=(0, 1), keepdims=True) - shifted_sum
    out_ref[...] = total.reshape(1, 1) / _N


def kernel(pred, target):
    t2 = target.reshape(_D, _D)
    p2 = pred.reshape(_D, _D)
    a = jnp.arange(_D)
    upper = (a[None, :] >= a[:, None]).astype(jnp.float32)
    strict = (a[:, None] > a[None, :]).astype(jnp.float32)
    out = pl.pallas_call(
        _listmle_body,
        out_shape=jax.ShapeDtypeStruct((1, 1), jnp.float32),
    )(t2, p2, upper, strict)
    return out[0, 0]
